# baseline (speedup 1.0000x reference)
import functools

import jax
import jax.numpy as jnp
from jax import lax
from jax.experimental import pallas as pl
from jax.experimental.pallas import tpu as pltpu
from jax.experimental.pallas import tpu_sc as plsc

N = 4096
NC = 2
NS = 16
L = 16
NW = NC * NS
STRIP = N // NW
NUM_CLASSES = 5


_DNUMS = lax.GatherDimensionNumbers(
    offset_dims=(), collapsed_slice_dims=(0,), start_index_map=(0,)
)


def _bcast(vec, lane):
    idx = jnp.full((L, 1), lane, jnp.int32)
    return lax.gather(
        vec, idx, _DNUMS, slice_sizes=(1,),
        mode=lax.GatherScatterMode.PROMISE_IN_BOUNDS,
    )


def _sc_body(p_hbm, c_hbm, out_hbm, p_v, c_v, res_v, sem):
    cid = lax.axis_index("c")
    sid = lax.axis_index("s")
    wid = sid * NC + cid
    pltpu.async_copy(p_hbm, p_v, sem).wait()
    pltpu.async_copy(c_hbm, c_v, sem).wait()
    base = wid * STRIP

    def i_body(i, acc):
        off = base + i
        blk = (off // L) * L
        lane = off - blk
        pi = _bcast(p_v[pl.ds(blk, L)], lane)
        ci = _bcast(c_v[pl.ds(blk, L)], lane)

        def j_body(j, a):
            pj = p_v[pl.ds(j * L, L)]
            cj = c_v[pl.ds(j * L, L)]
            return a + jnp.where(cj > ci, jnp.maximum(pi - pj, 0.0), 0.0)

        return lax.fori_loop(0, N // L, j_body, acc)

    acc = lax.fori_loop(0, STRIP, i_body, jnp.zeros((L,), jnp.float32))
    res_v[pl.ds(0, L)] = acc

    for a in range(NUM_CLASSES):
        h = jnp.zeros((L,), jnp.float32)
        for k in range(STRIP // L):
            cv = c_v[pl.ds(base + k * L, L)]
            h = h + jnp.where(cv == a, 1.0, 0.0)
        res_v[pl.ds((1 + a) * L, L)] = h

    pltpu.sync_copy(res_v, out_hbm.at[wid])


@jax.jit
def kernel(prediction, label):
    mesh = plsc.VectorSubcoreMesh(
        core_axis_name="c", subcore_axis_name="s", num_cores=NC, num_subcores=NS
    )
    parts = pl.kernel(
        _sc_body,
        out_type=jax.ShapeDtypeStruct((NW, (1 + NUM_CLASSES) * L), jnp.float32),
        mesh=mesh,
        scratch_types=[
            pltpu.VMEM((N,), jnp.float32),
            pltpu.VMEM((N,), jnp.int32),
            pltpu.VMEM(((1 + NUM_CLASSES) * L,), jnp.float32),
            pltpu.SemaphoreType.DMA,
        ],
    )(prediction, label.astype(jnp.int32))
    s = jnp.sum(parts[:, :L])
    hist = jnp.sum(parts[:, L:].reshape(NW, NUM_CLASSES, L), axis=(0, 2))
    n_pairs = (jnp.float32(N) * jnp.float32(N) - jnp.sum(hist * hist)) * 0.5
    return jnp.where(n_pairs > 0, s / n_pairs, jnp.float32(0.0))


# speedup vs baseline: 3.3498x; 3.3498x over previous
import functools

import jax
import jax.numpy as jnp
from jax import lax
from jax.experimental import pallas as pl
from jax.experimental.pallas import tpu as pltpu
from jax.experimental.pallas import tpu_sc as plsc

N = 4096
NC = 2
NS = 16
L = 16
NW = NC * NS
NV = N // L
NUM_CLASSES = 5
PAD = (NUM_CLASSES + 1) * L
BIG = 1e30

_DNUMS = lax.GatherDimensionNumbers(
    offset_dims=(), collapsed_slice_dims=(0,), start_index_map=(0,)
)


def _bcast(vec, lane):
    idx = jnp.full((L, 1), lane, jnp.int32)
    return lax.gather(
        vec, idx, _DNUMS, slice_sizes=(1,),
        mode=lax.GatherScatterMode.PROMISE_IN_BOUNDS,
    )


def _scalar(vec):
    return lax.squeeze(lax.slice(vec, (0,), (1,)), dimensions=(0,))


def _sc_body(p_hbm, c_hbm, out_hbm, p_v, c_v, p_si, p_sj, res_v, sem):
    cid = lax.axis_index("c")
    sid = lax.axis_index("s")
    wid = sid * NC + cid
    pltpu.async_copy(p_hbm, p_v, sem).wait()
    pltpu.async_copy(c_hbm, c_v, sem).wait()
    lanes = lax.iota(jnp.int32, L)

    def hist_body(k, nas):
        cv = c_v[pl.ds(k * L, L)]
        return tuple(
            nas[a] + plsc.all_reduce_population_count(cv == a)
            for a in range(NUM_CLASSES)
        )

    nas = lax.fori_loop(
        0, NV, hist_body, tuple(jnp.zeros((L,), jnp.int32) for _ in range(NUM_CLASSES))
    )

    pns = [((na + (L - 1)) // L) * L for na in nas]
    poff_v = [jnp.zeros((L,), jnp.int32)]
    for a in range(NUM_CLASSES):
        poff_v.append(poff_v[a] + pns[a])
    poff = [_scalar(v) for v in poff_v]

    negbig = jnp.full((L,), -BIG, jnp.float32)
    posbig = jnp.full((L,), BIG, jnp.float32)

    def fill_body(k, carry):
        p_si[pl.ds(k * L, L)] = negbig
        p_sj[pl.ds(k * L, L)] = posbig
        return carry

    lax.fori_loop(0, (N + PAD) // L, fill_body, 0)

    def compact_body(k, poss):
        pv = p_v[pl.ds(k * L, L)]
        cv = c_v[pl.ds(k * L, L)]
        out = []
        for a in range(NUM_CLASSES):
            m = cv == a
            plsc.store_compressed(p_si.at[pl.ds(poss[a], L)], pv, mask=m)
            plsc.store_compressed(p_sj.at[pl.ds(poss[a], L)], pv, mask=m)
            out.append(poss[a] + _scalar(plsc.all_reduce_population_count(m)))
        return tuple(out)

    lax.fori_loop(0, NV, compact_body, tuple(poff[:NUM_CLASSES]))

    jhi = poff[NUM_CLASSES] // L
    n_mine = (jhi - wid + (NW - 1)) // NW

    def iv_body(t, tot):
        v = wid + t * NW
        base = v * L
        piv = p_si[pl.ds(base, L)]
        acls = jnp.int32(0)
        for b in range(1, NUM_CLASSES):
            acls = acls + jnp.where(base >= poff[b], 1, 0).astype(jnp.int32)
        jstart = poff[NUM_CLASSES]
        for b in range(NUM_CLASSES - 1, 0, -1):
            jstart = jnp.where(acls == b - 1, poff[b], jstart)
        pis = [_bcast(piv, l) for l in range(L)]

        def j_body(j, accs):
            pj = p_sj[pl.ds(j * L, L)]
            return tuple(
                acc + jnp.maximum(pi - pj, 0.0) for acc, pi in zip(accs, pis)
            )

        accs = lax.fori_loop(
            jstart // L, jhi, j_body,
            tuple(jnp.zeros((L,), jnp.float32) for _ in range(L)),
        )
        for acc in accs:
            tot = tot + acc
        return tot

    total = lax.fori_loop(0, n_mine, iv_body, jnp.zeros((L,), jnp.float32))

    hist = jnp.zeros((L,), jnp.float32)
    for a in range(NUM_CLASSES):
        hist = jnp.where(lanes == a, nas[a].astype(jnp.float32), hist)

    res_v[pl.ds(0, L)] = total
    res_v[pl.ds(L, L)] = hist
    pltpu.sync_copy(res_v, out_hbm.at[wid])


@jax.jit
def kernel(prediction, label):
    mesh = plsc.VectorSubcoreMesh(
        core_axis_name="c", subcore_axis_name="s", num_cores=NC, num_subcores=NS
    )
    parts = pl.kernel(
        _sc_body,
        out_type=jax.ShapeDtypeStruct((NW, 2 * L), jnp.float32),
        mesh=mesh,
        compiler_params=pltpu.CompilerParams(needs_layout_passes=False),
        scratch_types=[
            pltpu.VMEM((N,), jnp.float32),
            pltpu.VMEM((N,), jnp.int32),
            pltpu.VMEM((N + PAD,), jnp.float32),
            pltpu.VMEM((N + PAD,), jnp.float32),
            pltpu.VMEM((2 * L,), jnp.float32),
            pltpu.SemaphoreType.DMA,
        ],
    )(prediction, label.astype(jnp.int32))
    s = jnp.sum(parts[:, :L])
    hist = parts[0, L:]
    n_pairs = (jnp.float32(N) * jnp.float32(N) - jnp.sum(hist * hist)) * 0.5
    return jnp.where(n_pairs > 0, s / n_pairs, jnp.float32(0.0))
